# Initial kernel scaffold; baseline (speedup 1.0000x reference)
import jax
import jax.numpy as jnp
from jax.experimental import pallas as pl


def kernel(node_feature, edge_index, edge_type, node_type, rel_W1, rel_b1, rel_W2, rel_b2, rel_W3, rel_b3, nu_W1, nu_b1, nu_W2, nu_b2, nu_W3, nu_b3):
    raise NotImplementedError("write your pallas kernel here")


# speedup vs baseline: 2.6871x; 2.6871x over previous
import functools

import jax
import jax.numpy as jnp
from jax import lax
from jax.experimental import pallas as pl
from jax.experimental.pallas import tpu as pltpu
from jax.experimental.pallas import tpu_sc as plsc

_N = 10000
_E = 320000
_D = 128
_R = 3
_H = 64

_NP = 10240
_NC = 2
_NT = 16
_ROWS_PER_SC = _R * _NP // _NC
_ROWS_PER_TILE = _ROWS_PER_SC // _NT
_EDGES_PER_TILE = _E // _NT
_CHUNK = 32
_NCHUNK = _EDGES_PER_TILE // _CHUNK

_BN = 400


def _msg_body(x_ref, w1_ref, b1_ref, w2_ref, b2_ref, w3_ref, b3_ref, out_ref):
    x = x_ref[...]
    h = jnp.dot(x, w1_ref[0], preferred_element_type=jnp.float32) + b1_ref[0]
    h = jnp.maximum(h, 0.0)
    h = jnp.dot(h, w2_ref[0], preferred_element_type=jnp.float32) + b2_ref[0]
    h = jnp.maximum(h, 0.0)
    m = jnp.dot(h, w3_ref[0], preferred_element_type=jnp.float32) + b3_ref[0]
    out_ref[0] = jnp.maximum(m, 0.0)


def _out_body(x_ref, agg_ref, nt_ref, w1_ref, b1_ref, w2_ref, b2_ref,
              w3_ref, b3_ref, out_ref):
    x = x_ref[...]
    h = jnp.dot(jnp.maximum(x, 0.0), w1_ref[0:_D, :],
                preferred_element_type=jnp.float32)
    h += jnp.dot(agg_ref[0], w1_ref[_D:2 * _D, :],
                 preferred_element_type=jnp.float32)
    h += jnp.dot(agg_ref[1], w1_ref[2 * _D:3 * _D, :],
                 preferred_element_type=jnp.float32)
    h += jnp.dot(agg_ref[2], w1_ref[3 * _D:4 * _D, :],
                 preferred_element_type=jnp.float32)
    h = jnp.maximum(h + b1_ref[...], 0.0)
    h = jnp.dot(h, w2_ref[...], preferred_element_type=jnp.float32)
    h = jnp.maximum(h + b2_ref[...], 0.0)
    o = jnp.dot(h, w3_ref[...], preferred_element_type=jnp.float32)
    o = o + b3_ref[...]
    nt = nt_ref[...]
    upd = (nt == 0) | (nt == 1)
    out_ref[...] = jnp.where(upd, o, x)


def _sc_body(m_hbm, gidx_hbm, sidx_hbm, zeros_hbm, out_hbm,
             gidx_c, sidx_c, rows_v, lidx_v, acc, sem):
    c = lax.axis_index("c")
    s = lax.axis_index("s")
    pltpu.sync_copy(zeros_hbm, acc.at[pl.ds(s * _ROWS_PER_TILE, _ROWS_PER_TILE)])
    plsc.subcore_barrier()

    ebase = s * _EDGES_PER_TILE
    lo = c * _ROWS_PER_SC
    dummy = _N + c * (_NP // 2)

    def chunk_body(j, carry):
        cb = ebase + j * _CHUNK
        pltpu.sync_copy(gidx_hbm.at[pl.ds(cb, _CHUNK)], gidx_c)
        pltpu.sync_copy(sidx_hbm.at[pl.ds(cb, _CHUNK)], sidx_c)
        for i in range(_CHUNK // 16):
            sv = sidx_c[pl.ds(i * 16, 16)]
            owned = (sv >= lo) & (sv < lo + _ROWS_PER_SC)
            lidx_v[pl.ds(i * 16, 16)] = jnp.where(owned, sv - lo, dummy)
        pltpu.async_copy(m_hbm.at[gidx_c], rows_v, sem).wait()
        pltpu.sync_copy(rows_v, acc.at[lidx_v], add=True)
        return carry

    lax.fori_loop(0, _NCHUNK, chunk_body, 0)
    plsc.subcore_barrier()
    pltpu.sync_copy(
        acc.at[pl.ds(s * _ROWS_PER_TILE, _ROWS_PER_TILE)],
        out_hbm.at[pl.ds(c * _ROWS_PER_SC + s * _ROWS_PER_TILE,
                         _ROWS_PER_TILE)])


def kernel(node_feature, edge_index, edge_type, node_type, rel_W1, rel_b1,
           rel_W2, rel_b2, rel_W3, rel_b3, nu_W1, nu_b1, nu_W2, nu_b2,
           nu_W3, nu_b3):
    mtab = pl.pallas_call(
        _msg_body,
        grid=(_R, _N // _BN),
        in_specs=[
            pl.BlockSpec((_BN, _D), lambda r, n: (n, 0)),
            pl.BlockSpec((1, _D, _H), lambda r, n: (r, 0, 0)),
            pl.BlockSpec((1, 1, _H), lambda r, n: (r, 0, 0)),
            pl.BlockSpec((1, _H, _H), lambda r, n: (r, 0, 0)),
            pl.BlockSpec((1, 1, _H), lambda r, n: (r, 0, 0)),
            pl.BlockSpec((1, _H, _D), lambda r, n: (r, 0, 0)),
            pl.BlockSpec((1, 1, _D), lambda r, n: (r, 0, 0)),
        ],
        out_specs=pl.BlockSpec((1, _BN, _D), lambda r, n: (r, n, 0)),
        out_shape=jax.ShapeDtypeStruct((_R, _N, _D), jnp.float32),
    )(node_feature, rel_W1, rel_b1[:, None], rel_W2, rel_b2[:, None],
      rel_W3, rel_b3[:, None])
    mflat = mtab.reshape(_R * _N, _D)

    gidx = edge_type * _N + edge_index[0]
    sidx = edge_type * _NP + edge_index[1]
    zeros = jnp.zeros((_ROWS_PER_TILE, _D), jnp.float32)

    sc_agg = pl.kernel(
        _sc_body,
        out_type=jax.ShapeDtypeStruct((_R * _NP, _D), jnp.float32),
        mesh=plsc.VectorSubcoreMesh(core_axis_name="c", subcore_axis_name="s"),
        scratch_types=[
            pltpu.VMEM((_CHUNK,), jnp.int32),
            pltpu.VMEM((_CHUNK,), jnp.int32),
            pltpu.VMEM((_CHUNK, _D), jnp.float32),
            pltpu.VMEM((_CHUNK,), jnp.int32),
            pltpu.VMEM_SHARED((_ROWS_PER_SC, _D), jnp.float32),
            pltpu.SemaphoreType.DMA,
        ],
    )
    agg = sc_agg(mflat, gidx, sidx, zeros).reshape(_R, _NP, _D)

    nt2 = node_type.reshape(_N, 1)
    out = pl.pallas_call(
        _out_body,
        grid=(_N // _BN,),
        in_specs=[
            pl.BlockSpec((_BN, _D), lambda n: (n, 0)),
            pl.BlockSpec((_R, _BN, _D), lambda n: (0, n, 0)),
            pl.BlockSpec((_BN, 1), lambda n: (n, 0)),
            pl.BlockSpec((4 * _D, _H), lambda n: (0, 0)),
            pl.BlockSpec((1, _H), lambda n: (0, 0)),
            pl.BlockSpec((_H, _H), lambda n: (0, 0)),
            pl.BlockSpec((1, _H), lambda n: (0, 0)),
            pl.BlockSpec((_H, _D), lambda n: (0, 0)),
            pl.BlockSpec((1, _D), lambda n: (0, 0)),
        ],
        out_specs=pl.BlockSpec((_BN, _D), lambda n: (n, 0)),
        out_shape=jax.ShapeDtypeStruct((_N, _D), jnp.float32),
    )(node_feature, agg, nt2, nu_W1, nu_b1[None], nu_W2, nu_b2[None],
      nu_W3, nu_b3[None])
    return out


# speedup vs baseline: 5.0347x; 1.8736x over previous
import jax
import jax.numpy as jnp
from jax import lax
from jax.experimental import pallas as pl
from jax.experimental.pallas import tpu as pltpu
from jax.experimental.pallas import tpu_sc as plsc

_N = 10000
_E = 320000
_D = 128
_R = 3
_H = 64

_NC = 2
_NT = 16
_ROWS_PER_SC = _R * _N // _NC
_ACC_ROWS = _ROWS_PER_SC + 8
_EDGES_PER_TILE = _E // _NT
_CHUNK = 32
_BLOCK = 800
_NBLOCK = _EDGES_PER_TILE // _BLOCK
_CPB = _BLOCK // _CHUNK
_ZROWS = 120
_NZB = _ROWS_PER_SC // _ZROWS

_BN = 400


def _msg_body(x_ref, w1_ref, b1_ref, w2_ref, b2_ref, w3_ref, b3_ref, out_ref):
    x = x_ref[...]
    h = jnp.dot(x, w1_ref[0], preferred_element_type=jnp.float32) + b1_ref[0]
    h = jnp.maximum(h, 0.0)
    h = jnp.dot(h, w2_ref[0], preferred_element_type=jnp.float32) + b2_ref[0]
    h = jnp.maximum(h, 0.0)
    m = jnp.dot(h, w3_ref[0], preferred_element_type=jnp.float32) + b3_ref[0]
    out_ref[0] = jnp.maximum(m, 0.0)


def _out_body(x_ref, agg_ref, nt_ref, w1_ref, b1_ref, w2_ref, b2_ref,
              w3_ref, b3_ref, out_ref):
    x = x_ref[...]
    h = jnp.dot(jnp.maximum(x, 0.0), w1_ref[0:_D, :],
                preferred_element_type=jnp.float32)
    h += jnp.dot(agg_ref[0], w1_ref[_D:2 * _D, :],
                 preferred_element_type=jnp.float32)
    h += jnp.dot(agg_ref[1], w1_ref[2 * _D:3 * _D, :],
                 preferred_element_type=jnp.float32)
    h += jnp.dot(agg_ref[2], w1_ref[3 * _D:4 * _D, :],
                 preferred_element_type=jnp.float32)
    h = jnp.maximum(h + b1_ref[...], 0.0)
    h = jnp.dot(h, w2_ref[...], preferred_element_type=jnp.float32)
    h = jnp.maximum(h + b2_ref[...], 0.0)
    o = jnp.dot(h, w3_ref[...], preferred_element_type=jnp.float32)
    o = o + b3_ref[...]
    nt = nt_ref[...]
    upd = (nt == 0) | (nt == 1)
    out_ref[...] = jnp.where(upd, o, x)


def _sc_body(m_hbm, gidx_hbm, sidx_hbm, zeros_hbm, out_hbm,
             gb, sb, rows0, rows1, lidx, acc, semg):
    c = lax.axis_index("c")
    s = lax.axis_index("s")
    nzb = (_NZB - s + _NT - 1) // _NT

    def zinit(i, carry):
        blk = s + i * _NT
        pltpu.sync_copy(zeros_hbm, acc.at[pl.ds(blk * _ZROWS, _ZROWS)])
        return carry

    lax.fori_loop(0, nzb, zinit, 0)
    plsc.subcore_barrier()

    ebase = s * _EDGES_PER_TILE
    lo = c * _ROWS_PER_SC

    def wait_g(rows):
        pltpu.make_async_copy(m_hbm.at[pl.ds(0, _CHUNK)], rows, semg).wait()

    def issue_g(j, rows):
        pltpu.async_copy(m_hbm.at[gb.at[pl.ds(j * _CHUNK, _CHUNK)]], rows,
                         semg)

    def scatter(j, rows):
        for i in range(_CHUNK // 16):
            sv = sb[pl.ds(j * _CHUNK + i * 16, 16)]
            owned = (sv >= lo) & (sv < lo + _ROWS_PER_SC)
            lidx[pl.ds(i * 16, 16)] = jnp.where(owned, sv - lo, _ROWS_PER_SC)
        pltpu.sync_copy(rows, acc.at[lidx], add=True)

    def block_body(b, carry):
        bb = ebase + b * _BLOCK
        pltpu.sync_copy(gidx_hbm.at[pl.ds(bb, _BLOCK)], gb)
        pltpu.sync_copy(sidx_hbm.at[pl.ds(bb, _BLOCK)], sb)
        issue_g(0, rows0)

        def pair(k, carry2):
            j = 2 * k
            wait_g(rows0)
            issue_g(j + 1, rows1)
            scatter(j, rows0)
            wait_g(rows1)
            issue_g(j + 2, rows0)
            scatter(j + 1, rows1)
            return carry2

        lax.fori_loop(0, (_CPB - 1) // 2, pair, 0)
        wait_g(rows0)
        scatter(_CPB - 1, rows0)
        return carry

    lax.fori_loop(0, _NBLOCK, block_body, 0)
    plsc.subcore_barrier()

    def rdout(i, carry):
        blk = s + i * _NT
        pltpu.sync_copy(
            acc.at[pl.ds(blk * _ZROWS, _ZROWS)],
            out_hbm.at[pl.ds(c * _ROWS_PER_SC + blk * _ZROWS, _ZROWS)])
        return carry

    lax.fori_loop(0, nzb, rdout, 0)


def kernel(node_feature, edge_index, edge_type, node_type, rel_W1, rel_b1,
           rel_W2, rel_b2, rel_W3, rel_b3, nu_W1, nu_b1, nu_W2, nu_b2,
           nu_W3, nu_b3):
    mtab = pl.pallas_call(
        _msg_body,
        grid=(_R, _N // _BN),
        in_specs=[
            pl.BlockSpec((_BN, _D), lambda r, n: (n, 0)),
            pl.BlockSpec((1, _D, _H), lambda r, n: (r, 0, 0)),
            pl.BlockSpec((1, 1, _H), lambda r, n: (r, 0, 0)),
            pl.BlockSpec((1, _H, _H), lambda r, n: (r, 0, 0)),
            pl.BlockSpec((1, 1, _H), lambda r, n: (r, 0, 0)),
            pl.BlockSpec((1, _H, _D), lambda r, n: (r, 0, 0)),
            pl.BlockSpec((1, 1, _D), lambda r, n: (r, 0, 0)),
        ],
        out_specs=pl.BlockSpec((1, _BN, _D), lambda r, n: (r, n, 0)),
        out_shape=jax.ShapeDtypeStruct((_R, _N, _D), jnp.float32),
    )(node_feature, rel_W1, rel_b1[:, None], rel_W2, rel_b2[:, None],
      rel_W3, rel_b3[:, None])
    mflat = mtab.reshape(_R * _N, _D)

    gidx = edge_type * _N + edge_index[0]
    sidx = edge_type * _N + edge_index[1]
    zeros = jnp.zeros((_ZROWS, _D), jnp.float32)

    sc_agg = pl.kernel(
        _sc_body,
        out_type=jax.ShapeDtypeStruct((_R * _N, _D), jnp.float32),
        mesh=plsc.VectorSubcoreMesh(core_axis_name="c", subcore_axis_name="s"),
        scratch_types=[
            pltpu.VMEM((_BLOCK,), jnp.int32),
            pltpu.VMEM((_BLOCK,), jnp.int32),
            pltpu.VMEM((_CHUNK, _D), jnp.float32),
            pltpu.VMEM((_CHUNK, _D), jnp.float32),
            pltpu.VMEM((_CHUNK,), jnp.int32),
            pltpu.VMEM_SHARED((_ACC_ROWS, _D), jnp.float32),
            pltpu.SemaphoreType.DMA,
        ],
    )
    agg = sc_agg(mflat, gidx, sidx, zeros).reshape(_R, _N, _D)

    nt2 = node_type.reshape(_N, 1)
    out = pl.pallas_call(
        _out_body,
        grid=(_N // _BN,),
        in_specs=[
            pl.BlockSpec((_BN, _D), lambda n: (n, 0)),
            pl.BlockSpec((_R, _BN, _D), lambda n: (0, n, 0)),
            pl.BlockSpec((_BN, 1), lambda n: (n, 0)),
            pl.BlockSpec((4 * _D, _H), lambda n: (0, 0)),
            pl.BlockSpec((1, _H), lambda n: (0, 0)),
            pl.BlockSpec((_H, _H), lambda n: (0, 0)),
            pl.BlockSpec((1, _H), lambda n: (0, 0)),
            pl.BlockSpec((_H, _D), lambda n: (0, 0)),
            pl.BlockSpec((1, _D), lambda n: (0, 0)),
        ],
        out_specs=pl.BlockSpec((_BN, _D), lambda n: (n, 0)),
        out_shape=jax.ShapeDtypeStruct((_N, _D), jnp.float32),
    )(node_feature, agg, nt2, nu_W1, nu_b1[None], nu_W2, nu_b2[None],
      nu_W3, nu_b3[None])
    return out
